# baseline (speedup 1.0000x reference)
import functools

import jax
import jax.numpy as jnp
from jax import lax
from jax.experimental import pallas as pl
from jax.experimental.pallas import tpu as pltpu
from jax.experimental.pallas import tpu_sc as plsc

_HIDDEN = 64
_MAX_LEN = 2048
_TABLE_ROWS = 2 * _MAX_LEN - 1


def _make_sc_kernel(S: int, D: int):
    info = plsc.get_sparse_core_info()
    num_cores, num_subcores = info.num_cores, info.num_subcores
    num_workers = num_cores * num_subcores
    rows_per_worker = S // num_workers

    half = S // 2
    span = half + rows_per_worker
    depth = 8

    mesh = plsc.VectorSubcoreMesh(core_axis_name="c", subcore_axis_name="s")

    @functools.partial(
        pl.kernel,
        mesh=mesh,
        out_type=jax.ShapeDtypeStruct((S, S, D), jnp.float32),
        scratch_types=[
            pltpu.VMEM((span, D), jnp.float32),
            pltpu.SemaphoreType.DMA,
        ],
        compiler_params=pltpu.CompilerParams(use_tc_tiling_on_sc=False),
    )
    def sc_kernel(table_hbm, out_hbm, buf, sem):
        c = lax.axis_index("c")
        s = lax.axis_index("s")
        wid = c * num_subcores + s
        base = wid * rows_per_worker

        for h in range(2):
            start = (S - 1) - (base + rows_per_worker - 1) + h * half
            pltpu.sync_copy(table_hbm.at[pl.ds(start, span)], buf)

            def copy_descr(k, _h=h):
                src = buf.at[pl.ds((rows_per_worker - 1) - k, half)]
                dst = out_hbm.at[base + k, pl.ds(_h * half, half)]
                return pltpu.make_async_copy(src, dst, sem)

            for t in range(depth):
                copy_descr(t).start()

            @pl.loop(0, rows_per_worker - depth)
            def _steady(k):
                copy_descr(k).wait()
                copy_descr(k + depth).start()

            @pl.loop(0, depth)
            def _drain(k):
                copy_descr(rows_per_worker - depth + k).wait()

    return sc_kernel


_sc_kernel = _make_sc_kernel(_MAX_LEN, _HIDDEN)


def kernel(x, positional_params):
    del x
    table = jnp.pad(positional_params, ((0, 1), (0, 0)))
    return _sc_kernel(table)


# speedup vs baseline: 1.2934x; 1.2934x over previous
import functools

import jax
import jax.numpy as jnp
from jax import lax
from jax.experimental import pallas as pl
from jax.experimental.pallas import tpu as pltpu
from jax.experimental.pallas import tpu_sc as plsc

_HIDDEN = 64
_MAX_LEN = 2048
_TABLE_ROWS = 2 * _MAX_LEN - 1


def _make_sc_kernel(S: int, D: int):
    info = plsc.get_sparse_core_info()
    num_cores, num_subcores = info.num_cores, info.num_subcores
    num_workers = num_cores * num_subcores
    rows_per_worker = S // num_workers

    half = S // 2
    span = half + rows_per_worker
    depth = 8

    mesh = plsc.VectorSubcoreMesh(core_axis_name="c", subcore_axis_name="s")

    @functools.partial(
        pl.kernel,
        mesh=mesh,
        out_type=jax.ShapeDtypeStruct((S, S, D), jnp.float32),
        scratch_types=[
            pltpu.VMEM((span, D), jnp.float32),
            pltpu.SemaphoreType.DMA,
        ],
        compiler_params=pltpu.CompilerParams(use_tc_tiling_on_sc=False),
    )
    def sc_kernel(table_hbm, out_hbm, buf, sem):
        c = lax.axis_index("c")
        s = lax.axis_index("s")
        wid = c * num_subcores + s
        base = wid * rows_per_worker

        for h in range(2):
            start = (S - 1) - (base + rows_per_worker - 1) + h * half
            pltpu.sync_copy(table_hbm.at[pl.ds(start, span)], buf)

            def copy_descr(k, _h=h):
                src = buf.at[pl.ds((rows_per_worker - 1) - k, half)]
                dst = out_hbm.at[base + k, pl.ds(_h * half, half)]
                return pltpu.make_async_copy(src, dst, sem)

            for t in range(depth):
                copy_descr(t).start()

            @pl.loop(0, rows_per_worker - depth)
            def _steady(k):
                copy_descr(k).wait()
                copy_descr(k + depth).start()

            @pl.loop(0, depth)
            def _drain(k):
                copy_descr(rows_per_worker - depth + k).wait()

    return sc_kernel


_sc_kernel = _make_sc_kernel(_MAX_LEN, _HIDDEN)


def _make_tc_kernel(S: int, D: int, block_rows: int = 8):
    T_pad = 2 * S

    def body(table_ref, out_ref):
        b = pl.program_id(0)
        for ii in range(block_rows):
            i = b * block_rows + ii
            out_ref[ii] = table_ref[pl.ds((S - 1) - i, S), :]

    return pl.pallas_call(
        body,
        grid=(S // block_rows,),
        in_specs=[pl.BlockSpec((T_pad, D), lambda b: (0, 0))],
        out_specs=pl.BlockSpec((block_rows, S, D), lambda b: (b, 0, 0)),
        out_shape=jax.ShapeDtypeStruct((S, S, D), jnp.float32),
    )


_tc_kernel = _make_tc_kernel(_MAX_LEN, _HIDDEN)


def kernel(x, positional_params):
    del x
    table = jnp.pad(positional_params, ((0, 1), (0, 0)))
    return _tc_kernel(table)
